# baseline (device time: 141150 ns/iter reference)
import jax
import jax.numpy as jnp
from jax import lax
from jax.experimental import pallas as pl
from jax.experimental.pallas import tpu as pltpu

N_Y = 4
H, Dh, Dr = 16, 128, 32
SCALE = (Dh + Dr) ** -0.5
BF = jnp.bfloat16
F32 = jnp.float32


def _proj_body(x_ref, wdkv_ref, wq_ref, wqr_ref, wkr_ref,
               xb_ref, c_ref, wqb_ref, wqrb_ref, wkrb_ref):
    xb = x_ref[:].astype(BF)
    xb_ref[:] = xb
    c_ref[:] = jnp.dot(xb, wdkv_ref[:].astype(BF),
                       preferred_element_type=F32).astype(BF)
    wqb_ref[:] = (wq_ref[:] * SCALE).astype(BF)
    wqrb_ref[:] = (wqr_ref[:] * SCALE).astype(BF)
    wkrb_ref[:] = wkr_ref[:].astype(BF)


def _gather_body(c_ref, wuk_ref, wuv_ref, xb_ref, wqb_ref, wqrb_ref,
                 wkrb_ref,
                 k_ref, v_ref, q_ref, qr_ref, kr_ref,
                 cf_ref, ukf_ref, uvf_ref,
                 ysend_sems, yrecv_sems, xsend_sems, xrecv_sems):
    ix = lax.axis_index("x")
    iy = lax.axis_index("y")
    iz = lax.axis_index("z")

    S, dc_sh = c_ref.shape
    half_S = S // 2
    half_r = dc_sh // 2

    barrier = pltpu.get_barrier_semaphore()

    @pl.when(iy > 0)
    def _():
        pl.semaphore_signal(barrier, inc=1, device_id=(ix, iy - 1, iz),
                            device_id_type=pl.DeviceIdType.MESH)

    @pl.when(iy < N_Y - 1)
    def _():
        pl.semaphore_signal(barrier, inc=1, device_id=(ix, iy + 1, iz),
                            device_id_type=pl.DeviceIdType.MESH)

    pl.semaphore_signal(barrier, inc=1, device_id=(1 - ix, iy, iz),
                        device_id_type=pl.DeviceIdType.MESH)

    @pl.when((iy == 0) | (iy == N_Y - 1))
    def _():
        pl.semaphore_wait(barrier, 2)

    @pl.when((iy > 0) & (iy < N_Y - 1))
    def _():
        pl.semaphore_wait(barrier, 3)

    cf_ref[:, pl.ds(iy * dc_sh, dc_sh)] = c_ref[:]
    ukf_ref[pl.ds(iy * dc_sh, dc_sh)] = wuk_ref[:].astype(BF)
    uvf_ref[pl.ds(iy * dc_sh, dc_sh)] = wuv_ref[:].astype(BF)

    def c_slice(ref, chunk, hx):
        return ref.at[pl.ds(hx * half_S, half_S),
                      pl.ds(chunk * dc_sh, dc_sh)]

    def w_slice(ref, chunk, hx):
        return ref.at[pl.ds(chunk * dc_sh + hx * half_r, half_r)]

    tensors = ((cf_ref, c_slice, 0), (ukf_ref, w_slice, 1),
               (uvf_ref, w_slice, 2))

    def desc(ref, slc, t, sems_pair, d, s, chunk, hx, target):
        send_sems, recv_sems = sems_pair
        return pltpu.make_async_remote_copy(
            src_ref=slc(ref, chunk, hx),
            dst_ref=slc(ref, chunk, hx),
            send_sem=send_sems.at[t, d, s],
            recv_sem=recv_sems.at[t, d, s],
            device_id=target,
            device_id_type=pl.DeviceIdType.MESH,
        )

    ysems = (ysend_sems, yrecv_sems)
    xsems = (xsend_sems, xrecv_sems)

    def ysend_cond(d, s):
        if d == 0:
            return (iy < N_Y - 1) & (iy >= s)
        return (iy > 0) & (iy + s <= N_Y - 1)

    def yrecv_cond(d, s):
        if d == 0:
            return (iy > 0) & (iy - 1 >= s)
        return (iy < N_Y - 1) & (iy + 1 + s <= N_Y - 1)

    def yrecv_chunk(d, s):
        return iy - 1 - s if d == 0 else iy + 1 + s

    def start_ysends(s):
        @pl.when(ysend_cond(0, s))
        def _():
            for ref, slc, t in tensors:
                desc(ref, slc, t, ysems, 0, s, iy - s, ix,
                     (ix, iy + 1, iz)).start()

        @pl.when(ysend_cond(1, s))
        def _():
            for ref, slc, t in tensors:
                desc(ref, slc, t, ysems, 1, s, iy + s, ix,
                     (ix, iy - 1, iz)).start()

    def wait_yrecvs(s):
        for d in (0, 1):
            @pl.when(yrecv_cond(d, s))
            def _(d=d):
                nbr = iy - 1 if d == 0 else iy + 1
                for ref, slc, t in tensors:
                    desc(ref, slc, t, ysems, d, s, yrecv_chunk(d, s), ix,
                         (ix, nbr, iz)).wait_recv()

    def start_xsends(s):
        for d in (0, 1):
            @pl.when(yrecv_cond(d, s))
            def _(d=d):
                for ref, slc, t in tensors:
                    desc(ref, slc, t, xsems, d, s, yrecv_chunk(d, s), ix,
                         (1 - ix, iy, iz)).start()

    def wait_xrecvs(s):
        for d in (0, 1):
            @pl.when(yrecv_cond(d, s))
            def _(d=d):
                for ref, slc, t in tensors:
                    desc(ref, slc, t, xsems, d, s, yrecv_chunk(d, s),
                         1 - ix, (1 - ix, iy, iz)).wait_recv()

    xb = xb_ref[:]
    D = xb.shape[1]
    qcol = D // 4

    def q_block(j):
        q_ref[:, pl.ds(j * qcol, qcol)] = jnp.dot(
            xb, wqb_ref[:, pl.ds(j * qcol, qcol)],
            preferred_element_type=F32).astype(BF)

    start_ysends(0)
    q_block(0)
    wait_yrecvs(0)
    start_ysends(1)
    start_xsends(0)
    q_block(1)
    wait_yrecvs(1)
    start_ysends(2)
    start_xsends(1)
    q_block(2)
    wait_yrecvs(2)
    start_xsends(2)
    q_block(3)
    qr = jnp.dot(xb, wqrb_ref[:], preferred_element_type=F32).astype(BF)
    n_heads = qr.shape[1] // Dr
    for h in range(n_heads):
        qr_ref[h] = qr[:, h * Dr:(h + 1) * Dr]
    kr_ref[:] = jnp.dot(xb, wkrb_ref[:],
                        preferred_element_type=F32).astype(BF)

    for s in range(N_Y - 1):
        wait_xrecvs(s)

    cf = cf_ref[:]
    k_ref[:] = jnp.dot(cf, ukf_ref[:],
                       preferred_element_type=F32).astype(BF)
    v_ref[:] = jnp.dot(cf, uvf_ref[:],
                       preferred_element_type=F32).astype(BF)

    for s in range(N_Y - 1):
        @pl.when(ysend_cond(0, s))
        def _(s=s):
            for ref, slc, t in tensors:
                desc(ref, slc, t, ysems, 0, s, iy - s, ix,
                     (ix, iy + 1, iz)).wait_send()

        @pl.when(ysend_cond(1, s))
        def _(s=s):
            for ref, slc, t in tensors:
                desc(ref, slc, t, ysems, 1, s, iy + s, ix,
                     (ix, iy - 1, iz)).wait_send()

        for d in (0, 1):
            @pl.when(yrecv_cond(d, s))
            def _(d=d, s=s):
                for ref, slc, t in tensors:
                    desc(ref, slc, t, xsems, d, s, yrecv_chunk(d, s), ix,
                         (1 - ix, iy, iz)).wait_send()


def _attn_body(q_ref, k_ref, v_ref, qr_ref, kr_ref, o_ref):
    qhat = jnp.concatenate([q_ref[:], qr_ref[0]], axis=1)
    khat = jnp.concatenate([k_ref[:], kr_ref[:]], axis=1)
    s = lax.dot_general(qhat, khat, (((1,), (1,)), ((), ())),
                        preferred_element_type=F32)
    p = jnp.exp(s.astype(BF))
    o = jnp.dot(p, v_ref[:], preferred_element_type=F32)
    l = jnp.dot(p, jnp.ones((p.shape[1], 8), BF),
                preferred_element_type=F32)
    o_ref[:] = (o / l[:, 0:1]).astype(BF)


def _out_body(o_ref, wo_ref, out_ref):
    out_ref[:] = jnp.dot(o_ref[:], wo_ref[:].astype(BF),
                         preferred_element_type=F32)


def kernel(x, Wdkv, Wuk, Wuv, Wq, Wqr, Wkr, Wo):
    B, S, D = x.shape
    dc_sh = Wdkv.shape[1]
    dc = N_Y * dc_sh
    x2 = x.reshape(S, D)
    vmem = pl.BlockSpec(memory_space=pltpu.VMEM)

    xb, c, Wqb, Wqrb, Wkrb = pl.pallas_call(
        _proj_body,
        out_shape=[
            jax.ShapeDtypeStruct((S, D), BF),
            jax.ShapeDtypeStruct((S, dc_sh), BF),
            jax.ShapeDtypeStruct((D, D), BF),
            jax.ShapeDtypeStruct((D, H * Dr), BF),
            jax.ShapeDtypeStruct((D, Dr), BF),
        ],
        in_specs=[vmem] * 5,
        out_specs=[vmem] * 5,
        compiler_params=pltpu.CompilerParams(
            vmem_limit_bytes=60 * 1024 * 1024),
    )(x2, Wdkv, Wq, Wqr, Wkr)

    K, V, Q, Qr, Kr = pl.pallas_call(
        _gather_body,
        out_shape=[
            jax.ShapeDtypeStruct((S, D), BF),
            jax.ShapeDtypeStruct((S, D), BF),
            jax.ShapeDtypeStruct((S, D), BF),
            jax.ShapeDtypeStruct((H, S, Dr), BF),
            jax.ShapeDtypeStruct((S, Dr), BF),
        ],
        in_specs=[vmem] * 7,
        out_specs=[vmem] * 5,
        scratch_shapes=[
            pltpu.VMEM((S, dc), BF),
            pltpu.VMEM((dc, D), BF),
            pltpu.VMEM((dc, D), BF),
            pltpu.SemaphoreType.DMA((3, 2, N_Y - 1)),
            pltpu.SemaphoreType.DMA((3, 2, N_Y - 1)),
            pltpu.SemaphoreType.DMA((3, 2, N_Y - 1)),
            pltpu.SemaphoreType.DMA((3, 2, N_Y - 1)),
        ],
        compiler_params=pltpu.CompilerParams(
            collective_id=0, vmem_limit_bytes=60 * 1024 * 1024),
    )(c, Wuk, Wuv, xb, Wqb, Wqrb, Wkrb)

    O = pl.pallas_call(
        _attn_body,
        grid=(H,),
        out_shape=jax.ShapeDtypeStruct((S, D), BF),
        in_specs=[
            pl.BlockSpec((S, Dh), lambda h: (0, h)),
            pl.BlockSpec((S, Dh), lambda h: (0, h)),
            pl.BlockSpec((S, Dh), lambda h: (0, h)),
            pl.BlockSpec((1, S, Dr), lambda h: (h, 0, 0)),
            pl.BlockSpec((S, Dr), lambda h: (0, 0)),
        ],
        out_specs=pl.BlockSpec((S, Dh), lambda h: (0, h)),
    )(Q, K, V, Qr, Kr)

    out = pl.pallas_call(
        _out_body,
        grid=(4,),
        out_shape=jax.ShapeDtypeStruct((S, D), F32),
        in_specs=[
            pl.BlockSpec((S, D), lambda j: (0, 0)),
            pl.BlockSpec((D, D // 4), lambda j: (0, j)),
        ],
        out_specs=pl.BlockSpec((S, D // 4), lambda j: (0, j)),
        compiler_params=pltpu.CompilerParams(
            vmem_limit_bytes=60 * 1024 * 1024),
    )(O, Wo)

    return out.reshape(B, S, D)


# device time: 137769 ns/iter; 1.0245x vs baseline; 1.0245x over previous
import jax
import jax.numpy as jnp
from jax import lax
from jax.experimental import pallas as pl
from jax.experimental.pallas import tpu as pltpu

N_Y = 4
H, Dh, Dr = 16, 128, 32
SCALE = (Dh + Dr) ** -0.5
BF = jnp.bfloat16
F32 = jnp.float32


def _proj_body(x_ref, wdkv_ref, wq_ref, wqr_ref, wkr_ref,
               xb_ref, c_ref, wqb_ref, wqrb_ref, wkrb_ref):
    xb = x_ref[:].astype(BF)
    xb_ref[:] = xb
    c_ref[:] = jnp.dot(xb, wdkv_ref[:].astype(BF),
                       preferred_element_type=F32).astype(BF)
    wqb_ref[:] = (wq_ref[:] * SCALE).astype(BF)
    wqrb_ref[:] = (wqr_ref[:] * SCALE).astype(BF)
    wkrb_ref[:] = wkr_ref[:].astype(BF)


def _gather_body(c_ref, wuk_ref, wuv_ref, xb_ref, wqb_ref, wqrb_ref,
                 wkrb_ref,
                 k_ref, v_ref, q_ref, qr_ref, kr_ref,
                 cf_ref, ukf_ref, uvf_ref,
                 ysend_sems, yrecv_sems, xsend_sems, xrecv_sems):
    ix = lax.axis_index("x")
    iy = lax.axis_index("y")
    iz = lax.axis_index("z")

    S, dc_sh = c_ref.shape
    half_S = S // 2
    half_r = dc_sh // 2

    barrier = pltpu.get_barrier_semaphore()

    @pl.when(iy > 0)
    def _():
        pl.semaphore_signal(barrier, inc=1, device_id=(ix, iy - 1, iz),
                            device_id_type=pl.DeviceIdType.MESH)

    @pl.when(iy < N_Y - 1)
    def _():
        pl.semaphore_signal(barrier, inc=1, device_id=(ix, iy + 1, iz),
                            device_id_type=pl.DeviceIdType.MESH)

    pl.semaphore_signal(barrier, inc=1, device_id=(1 - ix, iy, iz),
                        device_id_type=pl.DeviceIdType.MESH)

    @pl.when((iy == 0) | (iy == N_Y - 1))
    def _():
        pl.semaphore_wait(barrier, 2)

    @pl.when((iy > 0) & (iy < N_Y - 1))
    def _():
        pl.semaphore_wait(barrier, 3)

    cf_ref[:, pl.ds(iy * dc_sh, dc_sh)] = c_ref[:]
    ukf_ref[pl.ds(iy * dc_sh, dc_sh)] = wuk_ref[:].astype(BF)
    uvf_ref[pl.ds(iy * dc_sh, dc_sh)] = wuv_ref[:].astype(BF)

    def c_slice(ref, chunk, hx):
        return ref.at[pl.ds(hx * half_S, half_S),
                      pl.ds(chunk * dc_sh, dc_sh)]

    def w_slice(ref, chunk, hx):
        return ref.at[pl.ds(chunk * dc_sh + hx * half_r, half_r)]

    tensors = ((cf_ref, c_slice, 0), (ukf_ref, w_slice, 1),
               (uvf_ref, w_slice, 2))

    def desc(ref, slc, t, sems_pair, d, s, chunk, hx, target):
        send_sems, recv_sems = sems_pair
        return pltpu.make_async_remote_copy(
            src_ref=slc(ref, chunk, hx),
            dst_ref=slc(ref, chunk, hx),
            send_sem=send_sems.at[t, d, s],
            recv_sem=recv_sems.at[t, d, s],
            device_id=target,
            device_id_type=pl.DeviceIdType.MESH,
        )

    ysems = (ysend_sems, yrecv_sems)
    xsems = (xsend_sems, xrecv_sems)

    def ysend_cond(d, s):
        if d == 0:
            return (iy < N_Y - 1) & (iy >= s)
        return (iy > 0) & (iy + s <= N_Y - 1)

    def yrecv_cond(d, s):
        if d == 0:
            return (iy > 0) & (iy - 1 >= s)
        return (iy < N_Y - 1) & (iy + 1 + s <= N_Y - 1)

    def yrecv_chunk(d, s):
        return iy - 1 - s if d == 0 else iy + 1 + s

    def start_ysends(s):
        @pl.when(ysend_cond(0, s))
        def _():
            for ref, slc, t in tensors:
                desc(ref, slc, t, ysems, 0, s, iy - s, ix,
                     (ix, iy + 1, iz)).start()

        @pl.when(ysend_cond(1, s))
        def _():
            for ref, slc, t in tensors:
                desc(ref, slc, t, ysems, 1, s, iy + s, ix,
                     (ix, iy - 1, iz)).start()

    def wait_yrecvs(s):
        for d in (0, 1):
            @pl.when(yrecv_cond(d, s))
            def _(d=d):
                nbr = iy - 1 if d == 0 else iy + 1
                for ref, slc, t in tensors:
                    desc(ref, slc, t, ysems, d, s, yrecv_chunk(d, s), ix,
                         (ix, nbr, iz)).wait_recv()

    def start_xsends(s):
        for d in (0, 1):
            @pl.when(yrecv_cond(d, s))
            def _(d=d):
                for ref, slc, t in tensors:
                    desc(ref, slc, t, xsems, d, s, yrecv_chunk(d, s), ix,
                         (1 - ix, iy, iz)).start()

    def wait_xrecvs(s):
        for d in (0, 1):
            @pl.when(yrecv_cond(d, s))
            def _(d=d):
                for ref, slc, t in tensors:
                    desc(ref, slc, t, xsems, d, s, yrecv_chunk(d, s),
                         1 - ix, (1 - ix, iy, iz)).wait_recv()

    xb = xb_ref[:]
    D = xb.shape[1]
    qcol = D // 4

    def q_block(j):
        q_ref[:, pl.ds(j * qcol, qcol)] = jnp.dot(
            xb, wqb_ref[:, pl.ds(j * qcol, qcol)],
            preferred_element_type=F32).astype(BF)

    start_ysends(0)
    q_block(0)
    wait_yrecvs(0)
    start_ysends(1)
    start_xsends(0)
    q_block(1)
    wait_yrecvs(1)
    start_ysends(2)
    start_xsends(1)
    q_block(2)
    wait_yrecvs(2)
    start_xsends(2)
    q_block(3)
    qr_ref[:] = jnp.dot(xb, wqrb_ref[:],
                        preferred_element_type=F32).astype(BF)
    kr_ref[:] = jnp.dot(xb, wkrb_ref[:],
                        preferred_element_type=F32).astype(BF)

    for s in range(N_Y - 1):
        wait_xrecvs(s)

    cf = cf_ref[:]
    k_ref[:] = jnp.dot(cf, ukf_ref[:],
                       preferred_element_type=F32).astype(BF)
    v_ref[:] = jnp.dot(cf, uvf_ref[:],
                       preferred_element_type=F32).astype(BF)

    for s in range(N_Y - 1):
        @pl.when(ysend_cond(0, s))
        def _(s=s):
            for ref, slc, t in tensors:
                desc(ref, slc, t, ysems, 0, s, iy - s, ix,
                     (ix, iy + 1, iz)).wait_send()

        @pl.when(ysend_cond(1, s))
        def _(s=s):
            for ref, slc, t in tensors:
                desc(ref, slc, t, ysems, 1, s, iy + s, ix,
                     (ix, iy - 1, iz)).wait_send()

        for d in (0, 1):
            @pl.when(yrecv_cond(d, s))
            def _(d=d, s=s):
                for ref, slc, t in tensors:
                    desc(ref, slc, t, xsems, d, s, yrecv_chunk(d, s), ix,
                         (1 - ix, iy, iz)).wait_send()


def _attn_body(q_ref, k_ref, v_ref, qr_ref, kr_ref, o_ref):
    qhat = jnp.concatenate([q_ref[:], qr_ref[0]], axis=1)
    khat = jnp.concatenate([k_ref[:], kr_ref[:]], axis=1)
    s = lax.dot_general(qhat, khat, (((1,), (1,)), ((), ())),
                        preferred_element_type=F32)
    p = jnp.exp(s.astype(BF))
    o = jnp.dot(p, v_ref[:], preferred_element_type=F32)
    l = jnp.dot(p, jnp.ones(v_ref.shape, BF), preferred_element_type=F32)
    o_ref[:] = (o / l).astype(BF)


def _out_body(o_ref, wo_ref, out_ref):
    out_ref[:] = jnp.dot(o_ref[:], wo_ref[:].astype(BF),
                         preferred_element_type=F32)


def kernel(x, Wdkv, Wuk, Wuv, Wq, Wqr, Wkr, Wo):
    B, S, D = x.shape
    dc_sh = Wdkv.shape[1]
    dc = N_Y * dc_sh
    x2 = x.reshape(S, D)
    vmem = pl.BlockSpec(memory_space=pltpu.VMEM)

    xb, c, Wqb, Wqrb, Wkrb = pl.pallas_call(
        _proj_body,
        out_shape=[
            jax.ShapeDtypeStruct((S, D), BF),
            jax.ShapeDtypeStruct((S, dc_sh), BF),
            jax.ShapeDtypeStruct((D, D), BF),
            jax.ShapeDtypeStruct((D, H * Dr), BF),
            jax.ShapeDtypeStruct((D, Dr), BF),
        ],
        in_specs=[vmem] * 5,
        out_specs=[vmem] * 5,
        compiler_params=pltpu.CompilerParams(
            vmem_limit_bytes=60 * 1024 * 1024),
    )(x2, Wdkv, Wq, Wqr, Wkr)

    K, V, Q, Qr, Kr = pl.pallas_call(
        _gather_body,
        out_shape=[
            jax.ShapeDtypeStruct((S, D), BF),
            jax.ShapeDtypeStruct((S, D), BF),
            jax.ShapeDtypeStruct((S, D), BF),
            jax.ShapeDtypeStruct((S, H * Dr), BF),
            jax.ShapeDtypeStruct((S, Dr), BF),
        ],
        in_specs=[vmem] * 7,
        out_specs=[vmem] * 5,
        scratch_shapes=[
            pltpu.VMEM((S, dc), BF),
            pltpu.VMEM((dc, D), BF),
            pltpu.VMEM((dc, D), BF),
            pltpu.SemaphoreType.DMA((3, 2, N_Y - 1)),
            pltpu.SemaphoreType.DMA((3, 2, N_Y - 1)),
            pltpu.SemaphoreType.DMA((3, 2, N_Y - 1)),
            pltpu.SemaphoreType.DMA((3, 2, N_Y - 1)),
        ],
        compiler_params=pltpu.CompilerParams(
            collective_id=0, vmem_limit_bytes=60 * 1024 * 1024),
    )(c, Wuk, Wuv, xb, Wqb, Wqrb, Wkrb)

    Qr3 = Qr.reshape(S, H, Dr).transpose(1, 0, 2)

    O = pl.pallas_call(
        _attn_body,
        grid=(H,),
        out_shape=jax.ShapeDtypeStruct((S, D), BF),
        in_specs=[
            pl.BlockSpec((S, Dh), lambda h: (0, h)),
            pl.BlockSpec((S, Dh), lambda h: (0, h)),
            pl.BlockSpec((S, Dh), lambda h: (0, h)),
            pl.BlockSpec((1, S, Dr), lambda h: (h, 0, 0)),
            pl.BlockSpec((S, Dr), lambda h: (0, 0)),
        ],
        out_specs=pl.BlockSpec((S, Dh), lambda h: (0, h)),
    )(Q, K, V, Qr3, Kr)

    out = pl.pallas_call(
        _out_body,
        grid=(4,),
        out_shape=jax.ShapeDtypeStruct((S, D), F32),
        in_specs=[
            pl.BlockSpec((S, D), lambda j: (0, 0)),
            pl.BlockSpec((D, D // 4), lambda j: (0, j)),
        ],
        out_specs=pl.BlockSpec((S, D // 4), lambda j: (0, j)),
        compiler_params=pltpu.CompilerParams(
            vmem_limit_bytes=60 * 1024 * 1024),
    )(O, Wo)

    return out.reshape(B, S, D)
